# baseline (speedup 1.0000x reference)
import functools

import jax
import jax.numpy as jnp
from jax import lax
from jax.experimental import pallas as pl
from jax.experimental.pallas import tpu as pltpu
from jax.experimental.pallas import tpu_sc as plsc

K = 20
R = 256
PADD = 16
INF = float("inf")


def _topk_body(n_keys, xyz_ref, xyzT_ref, src_ref):
    q = xyz_ref[...]
    keys = xyzT_ref[...]
    xx = jnp.sum(q * q, axis=1, keepdims=True)
    yy = jnp.sum(keys * keys, axis=0, keepdims=True)
    inner = jnp.dot(q, keys, preferred_element_type=jnp.float32)
    dist = jnp.maximum(xx + yy - 2.0 * inner, 0.0)
    iota = lax.broadcasted_iota(jnp.int32, (R, n_keys), 1)
    big = jnp.int32(n_keys)
    idxs = []
    for j in range(K + 1):
        v = jnp.min(dist, axis=1, keepdims=True)
        eq = dist == v
        idx = jnp.min(jnp.where(eq, iota, big), axis=1, keepdims=True)
        if j > 0:
            idxs.append(idx)
        if j < K:
            dist = jnp.where(iota == idx, INF, dist)
    src_ref[...] = jnp.concatenate(idxs, axis=1)


def _topk_b(xyz_b, xyzT_b):
    N = xyz_b.shape[0]
    return pl.pallas_call(
        functools.partial(_topk_body, N),
        grid=(N // R,),
        in_specs=[
            pl.BlockSpec((R, 3), lambda i: (i, 0)),
            pl.BlockSpec((3, N), lambda i: (0, 0)),
        ],
        out_specs=pl.BlockSpec((R, K), lambda i: (i, 0)),
        out_shape=jax.ShapeDtypeStruct((N, K), jnp.int32),
    )(xyz_b, xyzT_b)


def _gather_b(pos_pad_b, idx_flat_b):
    E = idx_flat_b.shape[0]
    info = plsc.get_sparse_core_info()
    NC, NS = info.num_cores, info.num_subcores
    NW = NC * NS
    e_per_w = E // NW
    mesh = plsc.VectorSubcoreMesh(core_axis_name="c", subcore_axis_name="s")

    @functools.partial(
        pl.kernel, mesh=mesh,
        compiler_params=pltpu.CompilerParams(use_tc_tiling_on_sc=False),
        out_type=jax.ShapeDtypeStruct((E, PADD), jnp.float32),
        scratch_types=[
            pltpu.VMEM((e_per_w,), jnp.int32),
            pltpu.VMEM((e_per_w, PADD), jnp.float32),
            pltpu.SemaphoreType.DMA,
        ],
    )
    def k(pos_hbm, idx_hbm, out_hbm, idx_v, rows_v, sem):
        wid = lax.axis_index("s") * NC + lax.axis_index("c")
        base = wid * e_per_w
        pltpu.sync_copy(idx_hbm.at[pl.ds(base, e_per_w)], idx_v)
        pltpu.async_copy(pos_hbm.at[idx_v], rows_v, sem).wait()
        pltpu.sync_copy(rows_v, out_hbm.at[pl.ds(base, e_per_w)])

    return k(pos_pad_b, idx_flat_b)


def _edge_diff_body(xyz_ref, srows_ref, d_ref):
    q = xyz_ref[0]
    s = srows_ref[0]
    d_ref[0] = q[:, None, :] - s[:, :, 0:3]


def _edge_diff(xyz, srows4):
    B, N, _ = xyz.shape
    return pl.pallas_call(
        _edge_diff_body,
        grid=(B, N // R),
        in_specs=[
            pl.BlockSpec((1, R, 3), lambda b, i: (b, i, 0)),
            pl.BlockSpec((1, R, K, PADD), lambda b, i: (b, i, 0, 0)),
        ],
        out_specs=pl.BlockSpec((1, R, K, 3), lambda b, i: (b, i, 0, 0)),
        out_shape=jax.ShapeDtypeStruct((B, N, K, 3), jnp.float32),
    )(xyz, srows4)


def kernel(xyz):
    B, N, _ = xyz.shape
    pos = xyz
    xyzT = jnp.transpose(xyz, (0, 2, 1))
    pos_pad = jnp.pad(xyz, ((0, 0), (0, 0), (0, PADD - 3)))

    inds = []
    rows = []
    for b in range(B):
        ind_b = _topk_b(xyz[b], xyzT[b])
        inds.append(ind_b)
        rows.append(_gather_b(pos_pad[b], ind_b.reshape(N * K)))

    ind = jnp.stack(inds)
    src = ind.reshape(B, N * K)
    srows4 = jnp.stack(rows).reshape(B, N, K, PADD)
    d = _edge_diff(xyz, srows4).reshape(B, N * K, 3)

    dst = jnp.repeat(jnp.arange(N, dtype=jnp.int32), K)
    dst_b = jnp.tile(dst[None, :], (B, 1))
    f = jnp.ones((B, N, 1, 1), dtype=jnp.float32)
    return (pos, f, src, dst_b, d)


# speedup vs baseline: 1.0718x; 1.0718x over previous
import functools

import jax
import jax.numpy as jnp
from jax import lax
from jax.experimental import pallas as pl
from jax.experimental.pallas import tpu as pltpu
from jax.experimental.pallas import tpu_sc as plsc

K = 20
R = 256
PADD = 16
INF = float("inf")


def _topk_body(n_keys, xyz_ref, xyzT_ref, src_ref, srcoff_ref):
    b = pl.program_id(0)
    q = xyz_ref[0]
    keys = xyzT_ref[0]
    xx = jnp.sum(q * q, axis=1, keepdims=True)
    yy = jnp.sum(keys * keys, axis=0, keepdims=True)
    inner = jnp.dot(q, keys, preferred_element_type=jnp.float32)
    dist = jnp.maximum(xx + yy - 2.0 * inner, 0.0)
    iota = lax.broadcasted_iota(jnp.int32, (R, n_keys), 1)
    big = jnp.int32(n_keys)
    idxs = []
    for j in range(K + 1):
        v = jnp.min(dist, axis=1, keepdims=True)
        eq = dist == v
        idx = jnp.min(jnp.where(eq, iota, big), axis=1, keepdims=True)
        if j > 0:
            idxs.append(idx)
        if j < K:
            dist = jnp.where(iota == idx, INF, dist)
    ind = jnp.concatenate(idxs, axis=1)
    src_ref[0] = ind
    srcoff_ref[0] = ind + b * n_keys


def _topk(xyz, xyzT):
    B, N, _ = xyz.shape
    return pl.pallas_call(
        functools.partial(_topk_body, N),
        grid=(B, N // R),
        in_specs=[
            pl.BlockSpec((1, R, 3), lambda b, i: (b, i, 0)),
            pl.BlockSpec((1, 3, N), lambda b, i: (b, 0, 0)),
        ],
        out_specs=[
            pl.BlockSpec((1, R, K), lambda b, i: (b, i, 0)),
            pl.BlockSpec((1, R, K), lambda b, i: (b, i, 0)),
        ],
        out_shape=[
            jax.ShapeDtypeStruct((B, N, K), jnp.int32),
            jax.ShapeDtypeStruct((B, N, K), jnp.int32),
        ],
    )(xyz, xyzT)


def _gather(pos_pad, idx_flat):
    E = idx_flat.shape[0]
    info = plsc.get_sparse_core_info()
    NC, NS = info.num_cores, info.num_subcores
    NW = NC * NS
    e_per_w = E // NW
    mesh = plsc.VectorSubcoreMesh(core_axis_name="c", subcore_axis_name="s")

    @functools.partial(
        pl.kernel, mesh=mesh,
        compiler_params=pltpu.CompilerParams(use_tc_tiling_on_sc=False),
        out_type=jax.ShapeDtypeStruct((E, PADD), jnp.float32),
        scratch_types=[
            pltpu.VMEM((e_per_w,), jnp.int32),
            pltpu.VMEM((e_per_w, PADD), jnp.float32),
            pltpu.SemaphoreType.DMA,
        ],
    )
    def k(pos_hbm, idx_hbm, out_hbm, idx_v, rows_v, sem):
        wid = lax.axis_index("s") * NC + lax.axis_index("c")
        base = wid * e_per_w
        pltpu.sync_copy(idx_hbm.at[pl.ds(base, e_per_w)], idx_v)
        pltpu.async_copy(pos_hbm.at[idx_v], rows_v, sem).wait()
        pltpu.sync_copy(rows_v, out_hbm.at[pl.ds(base, e_per_w)])

    return k(pos_pad, idx_flat)


def _edge_diff_body(xyz_ref, srows_ref, d_ref):
    q = xyz_ref[0]
    s = srows_ref[0]
    d_ref[0] = q[:, None, :] - s[:, :, 0:3]


def _edge_diff(xyz, srows4):
    B, N, _ = xyz.shape
    return pl.pallas_call(
        _edge_diff_body,
        grid=(B, N // R),
        in_specs=[
            pl.BlockSpec((1, R, 3), lambda b, i: (b, i, 0)),
            pl.BlockSpec((1, R, K, PADD), lambda b, i: (b, i, 0, 0)),
        ],
        out_specs=pl.BlockSpec((1, R, K, 3), lambda b, i: (b, i, 0, 0)),
        out_shape=jax.ShapeDtypeStruct((B, N, K, 3), jnp.float32),
    )(xyz, srows4)


def kernel(xyz):
    B, N, _ = xyz.shape
    pos = xyz
    xyzT = jnp.transpose(xyz, (0, 2, 1))
    pos_pad = jnp.pad(xyz, ((0, 0), (0, 0), (0, PADD - 3))).reshape(B * N, PADD)

    ind, ind_off = _topk(xyz, xyzT)
    src = ind.reshape(B, N * K)

    srows = _gather(pos_pad, ind_off.reshape(B * N * K))
    srows4 = srows.reshape(B, N, K, PADD)
    d = _edge_diff(xyz, srows4).reshape(B, N * K, 3)

    dst = jnp.repeat(jnp.arange(N, dtype=jnp.int32), K)
    dst_b = jnp.tile(dst[None, :], (B, 1))
    f = jnp.ones((B, N, 1, 1), dtype=jnp.float32)
    return (pos, f, src, dst_b, d)


# speedup vs baseline: 1.1322x; 1.0564x over previous
import functools

import jax
import jax.numpy as jnp
from jax import lax
from jax.experimental import pallas as pl
from jax.experimental.pallas import tpu as pltpu
from jax.experimental.pallas import tpu_sc as plsc

K = 20
R = 256
PADD = 16
INF = float("inf")


def _topk_body(n_keys, xyz_ref, xyzT_ref, src_ref, srcoff_ref):
    b = pl.program_id(0)
    q = xyz_ref[0]
    keys = xyzT_ref[0]
    xx = jnp.sum(q * q, axis=1, keepdims=True)
    yy = jnp.sum(keys * keys, axis=0, keepdims=True)
    inner = jnp.dot(q, keys, preferred_element_type=jnp.float32)
    dist = jnp.maximum(xx + yy - 2.0 * inner, 0.0)
    iota = lax.broadcasted_iota(jnp.int32, (R, n_keys), 1)
    big = jnp.int32(n_keys)
    idxs = []
    for j in range(K + 1):
        v = jnp.min(dist, axis=1, keepdims=True)
        eq = dist == v
        idx = jnp.min(jnp.where(eq, iota, big), axis=1, keepdims=True)
        if j > 0:
            idxs.append(idx)
        if j < K:
            dist = jnp.where(iota == idx, INF, dist)
    ind = jnp.concatenate(idxs, axis=1)
    src_ref[0] = ind
    srcoff_ref[0] = ind + b * n_keys


def _topk(xyz):
    B, N, _ = xyz.shape
    xyzT = jnp.transpose(xyz, (0, 2, 1))
    return pl.pallas_call(
        functools.partial(_topk_body, N),
        grid=(B, N // R),
        in_specs=[
            pl.BlockSpec((1, R, 3), lambda b, i: (b, i, 0)),
            pl.BlockSpec((1, 3, N), lambda b, i: (b, 0, 0)),
        ],
        out_specs=[
            pl.BlockSpec((1, R, K), lambda b, i: (b, i, 0)),
            pl.BlockSpec((1, R, K), lambda b, i: (b, i, 0)),
        ],
        out_shape=[
            jax.ShapeDtypeStruct((B, N, K), jnp.int32),
            jax.ShapeDtypeStruct((B, N, K), jnp.int32),
        ],
    )(xyz, xyzT)


def _edge_gather(pos_pad, src_off, dst_off):
    E = src_off.shape[0]
    info = plsc.get_sparse_core_info()
    NC, NS = info.num_cores, info.num_subcores
    NW = NC * NS
    e_per_w = E // NW
    CH = 512
    n_ch = e_per_w // CH
    mesh = plsc.VectorSubcoreMesh(core_axis_name="c", subcore_axis_name="s")

    @functools.partial(
        pl.kernel, mesh=mesh,
        compiler_params=pltpu.CompilerParams(use_tc_tiling_on_sc=False),
        out_type=jax.ShapeDtypeStruct((E, PADD), jnp.float32),
        scratch_types=[
            pltpu.VMEM((CH,), jnp.int32),
            pltpu.VMEM((CH,), jnp.int32),
            pltpu.VMEM((CH, PADD), jnp.float32),
            pltpu.VMEM((CH, PADD), jnp.float32),
            pltpu.VMEM((CH, PADD), jnp.float32),
            pltpu.SemaphoreType.DMA,
        ],
    )
    def k(pos_hbm, sidx_hbm, didx_hbm, out_hbm,
          sidx_v, didx_v, srow_v, drow_v, out_v, sem):
        wid = lax.axis_index("s") * NC + lax.axis_index("c")

        def chunk_body(c, carry):
            base = wid * e_per_w + c * CH
            pltpu.sync_copy(sidx_hbm.at[pl.ds(base, CH)], sidx_v)
            pltpu.sync_copy(didx_hbm.at[pl.ds(base, CH)], didx_v)
            pltpu.async_copy(pos_hbm.at[sidx_v], srow_v, sem).wait()
            pltpu.async_copy(pos_hbm.at[didx_v], drow_v, sem).wait()

            def e_body(e, c2):
                out_v[e] = drow_v[e] - srow_v[e]
                return c2

            lax.fori_loop(0, CH, e_body, 0)
            pltpu.sync_copy(out_v, out_hbm.at[pl.ds(base, CH)])
            return carry

        lax.fori_loop(0, n_ch, chunk_body, 0)

    return k(pos_pad, src_off, dst_off)


def kernel(xyz):
    B, N, _ = xyz.shape
    pos = xyz
    ind, ind_off = _topk(xyz)
    src = ind.reshape(B, N * K)

    dst = jnp.repeat(jnp.arange(N, dtype=jnp.int32), K)
    dst_b = jnp.tile(dst[None, :], (B, 1))

    pos_pad = jnp.pad(pos.reshape(B * N, 3), ((0, 0), (0, PADD - 3)))
    src_off = ind_off.reshape(B * N * K)
    dst_off = (dst_b + jnp.arange(B, dtype=jnp.int32)[:, None] * N).reshape(-1)
    d_pad = _edge_gather(pos_pad, src_off, dst_off)
    d = d_pad[:, :3].reshape(B, N * K, 3)

    f = jnp.ones((B, N, 1, 1), dtype=jnp.float32)
    return (pos, f, src, dst_b, d)
